# baseline (speedup 1.0000x reference)
import functools

import jax
import jax.numpy as jnp
from jax import lax
from jax.experimental import pallas as pl
from jax.experimental.pallas import tpu as pltpu
from jax.experimental.pallas import tpu_sc as plsc

N = 10000
D = 128
H = 128
C = 10
BN_EPS = 1e-5

NC = 2
NS = 16
NW = NC * NS
CHUNK = 128
N_ACC = 10240
ROWS_PER_TILE = N_ACC // NS


def _seg_sum_body(ept, h_hbm, src_hbm, dst_hbm, zeros_hbm, agg_hbm,
                  acc, src_v, dst_v, rows_v, sem):
    c = lax.axis_index("c")
    s = lax.axis_index("s")
    wid = c * NS + s
    r0 = s * ROWS_PER_TILE

    pltpu.sync_copy(zeros_hbm.at[pl.ds(r0, ROWS_PER_TILE), :],
                    acc.at[pl.ds(r0, ROWS_PER_TILE), :])
    plsc.subcore_barrier()

    base = wid * ept
    nchunks = ept // CHUNK

    def body(i, carry):
        off = base + i * CHUNK
        pltpu.sync_copy(src_hbm.at[pl.ds(off, CHUNK)], src_v)
        pltpu.sync_copy(dst_hbm.at[pl.ds(off, CHUNK)], dst_v)
        pltpu.async_copy(h_hbm.at[src_v], rows_v, sem).wait()
        pltpu.sync_copy(rows_v, acc.at[dst_v], add=True)
        return carry

    lax.fori_loop(0, nchunks, body, 0)
    plsc.subcore_barrier()

    pltpu.sync_copy(acc.at[pl.ds(r0, ROWS_PER_TILE), :],
                    agg_hbm.at[c, pl.ds(r0, ROWS_PER_TILE), :])


@functools.lru_cache(maxsize=None)
def _make_seg_sum(ept):
    mesh = plsc.VectorSubcoreMesh(core_axis_name="c", subcore_axis_name="s")
    return pl.kernel(
        functools.partial(_seg_sum_body, ept),
        out_type=jax.ShapeDtypeStruct((NC, N_ACC, D), jnp.float32),
        mesh=mesh,
        scratch_types=[
            pltpu.VMEM_SHARED((N_ACC, D), jnp.float32),
            pltpu.VMEM((CHUNK,), jnp.int32),
            pltpu.VMEM((CHUNK,), jnp.int32),
            pltpu.VMEM((CHUNK, D), jnp.float32),
            pltpu.SemaphoreType.DMA,
        ],
    )


def _mlp_body(x_ref, agg_ref, w1_ref, b1_ref, g_ref, be_ref, w2_ref, b2_ref,
              out_ref):
    y = x_ref[...] + agg_ref[0, :N, :] + agg_ref[1, :N, :]
    z = lax.dot_general(y, w1_ref[...], (((1,), (1,)), ((), ())),
                        preferred_element_type=jnp.float32) + b1_ref[...]
    mean = jnp.mean(z, axis=0, keepdims=True)
    var = jnp.mean(z * z, axis=0, keepdims=True) - mean * mean
    inv = g_ref[...] * lax.rsqrt(var + BN_EPS)
    h = jnp.maximum((z - mean) * inv + be_ref[...], 0.0)
    o = lax.dot_general(h, w2_ref[...], (((1,), (1,)), ((), ())),
                        preferred_element_type=jnp.float32) + b2_ref[...]
    out_ref[...] = jnp.maximum(o, 0.0)


def _mlp(x, agg, w1, b1, gamma, beta, w2, b2):
    return pl.pallas_call(
        _mlp_body,
        out_shape=jax.ShapeDtypeStruct((N, H), jnp.float32),
    )(x, agg, w1, b1.reshape(1, H), gamma.reshape(1, H), beta.reshape(1, H),
      w2, b2.reshape(1, H))


CPAD = 16


def _head_body(x_ref, w1_ref, b1_ref, w2_ref, b2_ref, out_ref):
    t = lax.dot_general(x_ref[...], w1_ref[...], (((1,), (1,)), ((), ())),
                        preferred_element_type=jnp.float32) + b1_ref[...]
    t = jnp.maximum(t, 0.0)
    logits = lax.dot_general(t, w2_ref[...], (((1,), (1,)), ((), ())),
                             preferred_element_type=jnp.float32) + b2_ref[...]
    m = jnp.max(logits, axis=1, keepdims=True)
    lse = m + jnp.log(jnp.sum(jnp.exp(logits - m), axis=1, keepdims=True))
    out_ref[...] = logits - lse


def _head(x, w1, b1, w2, b2):
    w2p = jnp.zeros((CPAD, H), jnp.float32).at[:C].set(w2)
    b2p = jnp.full((CPAD,), -1e30, jnp.float32).at[:C].set(b2)
    out = pl.pallas_call(
        _head_body,
        out_shape=jax.ShapeDtypeStruct((N, CPAD), jnp.float32),
    )(x, w1, b1.reshape(1, H), w2p, b2p.reshape(1, CPAD))
    return out[:, :C]


def kernel(x, edge_index, convs_W1, convs_b1, convs_gamma, convs_beta,
           convs_W2, convs_b2, lin1_W, lin1_b, lin2_W, lin2_b):
    e = edge_index.shape[1]
    ept = -(-e // (NW * CHUNK * 2)) * CHUNK * 2
    e_pad = ept * NW
    src = jnp.concatenate(
        [edge_index[0], jnp.zeros((e_pad - e,), jnp.int32)])
    dst = jnp.concatenate(
        [edge_index[1], jnp.full((e_pad - e,), N, jnp.int32)])
    zeros = jnp.zeros((N_ACC, D), jnp.float32)

    seg_sum = _make_seg_sum(ept)
    h = x
    for i in range(5):
        agg = seg_sum(h, src, dst, zeros)
        h = _mlp(h, agg, convs_W1[i], convs_b1[i], convs_gamma[i],
                 convs_beta[i], convs_W2[i], convs_b2[i])
    return _head(h, lin1_W, lin1_b, lin2_W, lin2_b)


# speedup vs baseline: 2.1809x; 2.1809x over previous
import functools

import jax
import jax.numpy as jnp
from jax import lax
from jax.experimental import pallas as pl
from jax.experimental.pallas import tpu as pltpu
from jax.experimental.pallas import tpu_sc as plsc

N = 10000
D = 128
H = 128
C = 10
BN_EPS = 1e-5

NC = 2
NS = 16
NW = NC * NS
CHUNK = 128
N_ACC = 10240
ROWS_PER_TILE = N_ACC // NS


def _seg_sum_body(ept, h_hbm, src_hbm, dst_hbm, zeros_hbm, agg_hbm,
                  acc, src_v, dst_v, rows_v, sem):
    c = lax.axis_index("c")
    s = lax.axis_index("s")
    wid = c * NS + s
    r0 = s * ROWS_PER_TILE

    pltpu.sync_copy(zeros_hbm.at[pl.ds(r0, ROWS_PER_TILE), :],
                    acc.at[pl.ds(r0, ROWS_PER_TILE), :])
    plsc.subcore_barrier()

    base = wid * ept
    nchunks = ept // CHUNK

    def body(i, carry):
        off = base + i * CHUNK
        pltpu.sync_copy(src_hbm.at[pl.ds(off, CHUNK)], src_v)
        pltpu.sync_copy(dst_hbm.at[pl.ds(off, CHUNK)], dst_v)
        pltpu.async_copy(h_hbm.at[src_v], rows_v, sem).wait()
        pltpu.sync_copy(rows_v, acc.at[dst_v], add=True)
        return carry

    lax.fori_loop(0, nchunks, body, 0)
    plsc.subcore_barrier()

    pltpu.sync_copy(acc.at[pl.ds(r0, ROWS_PER_TILE), :],
                    agg_hbm.at[c, pl.ds(r0, ROWS_PER_TILE), :])


@functools.lru_cache(maxsize=None)
def _make_seg_sum(ept):
    mesh = plsc.VectorSubcoreMesh(core_axis_name="c", subcore_axis_name="s")
    return pl.kernel(
        functools.partial(_seg_sum_body, ept),
        out_type=jax.ShapeDtypeStruct((NC, N_ACC, D), jnp.float32),
        mesh=mesh,
        scratch_types=[
            pltpu.VMEM_SHARED((N_ACC, D), jnp.float32),
            pltpu.VMEM((CHUNK,), jnp.int32),
            pltpu.VMEM((CHUNK,), jnp.int32),
            pltpu.VMEM((CHUNK, D), jnp.float32),
            pltpu.SemaphoreType.DMA,
        ],
    )


def _mlp_body(x_ref, agg_ref, w1_ref, b1_ref, g_ref, be_ref, w2_ref, b2_ref,
              out_ref):
    y = x_ref[...] + agg_ref[0, :N, :] + agg_ref[1, :N, :]
    z = lax.dot_general(y, w1_ref[...], (((1,), (1,)), ((), ())),
                        preferred_element_type=jnp.float32) + b1_ref[...]
    mean = jnp.mean(z, axis=0, keepdims=True)
    var = jnp.mean(z * z, axis=0, keepdims=True) - mean * mean
    inv = g_ref[...] * lax.rsqrt(var + BN_EPS)
    h = jnp.maximum((z - mean) * inv + be_ref[...], 0.0)
    o = lax.dot_general(h, w2_ref[...], (((1,), (1,)), ((), ())),
                        preferred_element_type=jnp.float32) + b2_ref[...]
    out_ref[...] = jnp.maximum(o, 0.0)


def _mlp(x, agg, w1, b1, gamma, beta, w2, b2):
    return pl.pallas_call(
        _mlp_body,
        out_shape=jax.ShapeDtypeStruct((N, H), jnp.float32),
    )(x, agg, w1, b1.reshape(1, H), gamma.reshape(1, H), beta.reshape(1, H),
      w2, b2.reshape(1, H))


CPAD = 16


def _head_body(x_ref, w1_ref, b1_ref, w2_ref, b2_ref, out_ref):
    t = lax.dot_general(x_ref[...], w1_ref[...], (((1,), (1,)), ((), ())),
                        preferred_element_type=jnp.float32) + b1_ref[...]
    t = jnp.maximum(t, 0.0)
    logits = lax.dot_general(t, w2_ref[...], (((1,), (1,)), ((), ())),
                             preferred_element_type=jnp.float32) + b2_ref[...]
    m = jnp.max(logits, axis=1, keepdims=True)
    lse = m + jnp.log(jnp.sum(jnp.exp(logits - m), axis=1, keepdims=True))
    out_ref[...] = logits - lse


def _head(x, w1, b1, w2, b2):
    w2p = jnp.zeros((CPAD, H), jnp.float32).at[:C].set(w2)
    b2p = jnp.full((CPAD,), -1e30, jnp.float32).at[:C].set(b2)
    out = pl.pallas_call(
        _head_body,
        out_shape=jax.ShapeDtypeStruct((N, CPAD), jnp.float32),
    )(x, w1, b1.reshape(1, H), w2p, b2p.reshape(1, CPAD))
    return out[:, :C]


def kernel(x, edge_index, convs_W1, convs_b1, convs_gamma, convs_beta,
           convs_W2, convs_b2, lin1_W, lin1_b, lin2_W, lin2_b):
    e = edge_index.shape[1]
    ept = -(-e // (NW * CHUNK)) * CHUNK
    e_pad = ept * NW
    fill = jnp.arange(e_pad - e, dtype=jnp.int32)
    src = jnp.concatenate([edge_index[0], fill % N])
    dst = jnp.concatenate([edge_index[1], N + fill % (N_ACC - N)])
    zeros = jnp.zeros((N_ACC, D), jnp.float32)

    seg_sum = _make_seg_sum(ept)
    h = x
    for i in range(5):
        agg = seg_sum(h, src, dst, zeros)
        h = _mlp(h, agg, convs_W1[i], convs_b1[i], convs_gamma[i],
                 convs_beta[i], convs_W2[i], convs_b2[i])
    return _head(h, lin1_W, lin1_b, lin2_W, lin2_b)


# speedup vs baseline: 3.4128x; 1.5649x over previous
import functools

import jax
import jax.numpy as jnp
from jax import lax
from jax.experimental import pallas as pl
from jax.experimental.pallas import tpu as pltpu
from jax.experimental.pallas import tpu_sc as plsc

N = 10000
D = 128
H = 128
C = 10
BN_EPS = 1e-5

NC = 2
NS = 16
NW = NC * NS
CHUNK = 128
N_ACC = 10240
ROWS_PER_TILE = N_ACC // NS


def _seg_sum_body(ept, h_hbm, src_hbm, dst_hbm, zeros_hbm, agg_hbm,
                  acc, src0, dst0, src1, dst1, rows0, rows1, gsem0, gsem1):
    c = lax.axis_index("c")
    s = lax.axis_index("s")
    wid = c * NS + s
    r0 = s * ROWS_PER_TILE
    srcb = (src0, src1)
    dstb = (dst0, dst1)
    rows = (rows0, rows1)
    gsems = (gsem0, gsem1)

    pltpu.sync_copy(zeros_hbm.at[pl.ds(r0, ROWS_PER_TILE), :],
                    acc.at[pl.ds(r0, ROWS_PER_TILE), :])
    plsc.subcore_barrier()

    base = wid * ept
    nchunks = ept // CHUNK

    pltpu.sync_copy(src_hbm.at[pl.ds(base, CHUNK)], src0)
    pltpu.sync_copy(dst_hbm.at[pl.ds(base, CHUNK)], dst0)
    pltpu.async_copy(h_hbm.at[src0], rows0, gsem0)

    def pair(p, carry):
        for b in (0, 1):
            i = p * 2 + b
            nb = 1 - b

            @pl.when(i + 1 < nchunks)
            def _():
                off = base + (i + 1) * CHUNK
                pltpu.sync_copy(src_hbm.at[pl.ds(off, CHUNK)], srcb[nb])
                pltpu.sync_copy(dst_hbm.at[pl.ds(off, CHUNK)], dstb[nb])
                pltpu.async_copy(h_hbm.at[srcb[nb]], rows[nb], gsems[nb])

            pltpu.make_async_copy(h_hbm.at[srcb[b]], rows[b], gsems[b]).wait()
            pltpu.sync_copy(rows[b], acc.at[dstb[b]], add=True)
        return carry

    lax.fori_loop(0, nchunks // 2, pair, 0)
    plsc.subcore_barrier()

    pltpu.sync_copy(acc.at[pl.ds(r0, ROWS_PER_TILE), :],
                    agg_hbm.at[c, pl.ds(r0, ROWS_PER_TILE), :])


@functools.lru_cache(maxsize=None)
def _make_seg_sum(ept):
    mesh = plsc.VectorSubcoreMesh(core_axis_name="c", subcore_axis_name="s")
    return pl.kernel(
        functools.partial(_seg_sum_body, ept),
        out_type=jax.ShapeDtypeStruct((NC, N_ACC, D), jnp.float32),
        mesh=mesh,
        scratch_types=[
            pltpu.VMEM_SHARED((N_ACC, D), jnp.float32),
            pltpu.VMEM((CHUNK,), jnp.int32),
            pltpu.VMEM((CHUNK,), jnp.int32),
            pltpu.VMEM((CHUNK,), jnp.int32),
            pltpu.VMEM((CHUNK,), jnp.int32),
            pltpu.VMEM((CHUNK, D), jnp.float32),
            pltpu.VMEM((CHUNK, D), jnp.float32),
            pltpu.SemaphoreType.DMA,
            pltpu.SemaphoreType.DMA,
        ],
    )


def _mlp_body(x_ref, agg_ref, w1_ref, b1_ref, g_ref, be_ref, w2_ref, b2_ref,
              out_ref):
    y = x_ref[...] + agg_ref[0, :N, :] + agg_ref[1, :N, :]
    z = lax.dot_general(y, w1_ref[...], (((1,), (1,)), ((), ())),
                        preferred_element_type=jnp.float32) + b1_ref[...]
    mean = jnp.mean(z, axis=0, keepdims=True)
    var = jnp.mean(z * z, axis=0, keepdims=True) - mean * mean
    inv = g_ref[...] * lax.rsqrt(var + BN_EPS)
    h = jnp.maximum((z - mean) * inv + be_ref[...], 0.0)
    o = lax.dot_general(h, w2_ref[...], (((1,), (1,)), ((), ())),
                        preferred_element_type=jnp.float32) + b2_ref[...]
    out_ref[...] = jnp.maximum(o, 0.0)


def _mlp(x, agg, w1, b1, gamma, beta, w2, b2):
    return pl.pallas_call(
        _mlp_body,
        out_shape=jax.ShapeDtypeStruct((N, H), jnp.float32),
    )(x, agg, w1, b1.reshape(1, H), gamma.reshape(1, H), beta.reshape(1, H),
      w2, b2.reshape(1, H))


CPAD = 16


def _head_body(x_ref, w1_ref, b1_ref, w2_ref, b2_ref, out_ref):
    t = lax.dot_general(x_ref[...], w1_ref[...], (((1,), (1,)), ((), ())),
                        preferred_element_type=jnp.float32) + b1_ref[...]
    t = jnp.maximum(t, 0.0)
    logits = lax.dot_general(t, w2_ref[...], (((1,), (1,)), ((), ())),
                             preferred_element_type=jnp.float32) + b2_ref[...]
    m = jnp.max(logits, axis=1, keepdims=True)
    lse = m + jnp.log(jnp.sum(jnp.exp(logits - m), axis=1, keepdims=True))
    out_ref[...] = logits - lse


def _head(x, w1, b1, w2, b2):
    w2p = jnp.zeros((CPAD, H), jnp.float32).at[:C].set(w2)
    b2p = jnp.full((CPAD,), -1e30, jnp.float32).at[:C].set(b2)
    out = pl.pallas_call(
        _head_body,
        out_shape=jax.ShapeDtypeStruct((N, CPAD), jnp.float32),
    )(x, w1, b1.reshape(1, H), w2p, b2p.reshape(1, CPAD))
    return out[:, :C]


def kernel(x, edge_index, convs_W1, convs_b1, convs_gamma, convs_beta,
           convs_W2, convs_b2, lin1_W, lin1_b, lin2_W, lin2_b):
    e = edge_index.shape[1]
    ept = -(-e // (NW * CHUNK * 2)) * CHUNK * 2
    e_pad = ept * NW
    fill = jnp.arange(e_pad - e, dtype=jnp.int32)
    src = jnp.concatenate([edge_index[0], fill % N])
    dst = jnp.concatenate([edge_index[1], N + fill % (N_ACC - N)])
    zeros = jnp.zeros((N_ACC, D), jnp.float32)

    seg_sum = _make_seg_sum(ept)
    h = x
    for i in range(5):
        agg = seg_sum(h, src, dst, zeros)
        h = _mlp(h, agg, convs_W1[i], convs_b1[i], convs_gamma[i],
                 convs_beta[i], convs_W2[i], convs_b2[i])
    return _head(h, lin1_W, lin1_b, lin2_W, lin2_b)
